# baseline (speedup 1.0000x reference)
import jax
import jax.numpy as jnp
from jax import lax
from jax.experimental import pallas as pl
from jax.experimental.pallas import tpu as pltpu
from jax.experimental.pallas import tpu_sc as plsc

_GRID = 8
_DIM = 3
_A = 16
_S = _GRID ** 3
_NW = 32
_CH = 3136
_P = 100000
_TAIL = _P - (_NW - 1) * _CH
_RB = 2000


def _batcher_pairs(n):
    pairs = []
    p = 1
    while p < n:
        k = p
        while k >= 1:
            for j in range(k % p, n - k, 2 * k):
                for i in range(0, min(k, n - j - k)):
                    if (i + j) // (2 * p) == (i + j + k) // (2 * p):
                        pairs.append((i + j, i + j + k))
            k //= 2
        p *= 2
    return pairs


_BATCHER16 = _batcher_pairs(16)


def _grid_init_kernel(pts_ref, px_ref, mn_ref, mx_ref):
    pid = pl.program_id(0)
    mn = jnp.min(pts_ref[...], axis=0, keepdims=True)
    mx = jnp.max(pts_ref[...], axis=0, keepdims=True)

    @pl.when(pid == 0)
    def _():
        mn_ref[...] = mn
        mx_ref[...] = mx

    @pl.when(pid != 0)
    def _():
        mn_ref[...] = jnp.minimum(mn_ref[...], mn)
        mx_ref[...] = jnp.maximum(mx_ref[...], mx)

    @pl.when(pid == pl.num_programs(0) - 1)
    def _():
        ge = (mx_ref[...] - mn_ref[...]) / jnp.float32(_GRID) * jnp.float32(0.5)
        s = lax.broadcasted_iota(jnp.int32, (_S, _DIM), 0)
        c = lax.broadcasted_iota(jnp.int32, (_S, _DIM), 1)
        mesh = jnp.where(c == 0, s // (_GRID * _GRID),
                         jnp.where(c == 1, (s // _GRID) % _GRID, s % _GRID))
        mesh_ph = mesh.astype(jnp.float32) + jnp.float32(0.5)
        px_ref[...] = mesh_ph * ge * jnp.float32(2.0) + mn_ref[...]


def _sc_knn_body(xh, yh, zh, ph, out_h, xv, yv, zv, pv, ov):
    f32 = jnp.float32
    i32 = jnp.int32
    u32 = jnp.uint32
    wid = lax.axis_index("c") * 16 + lax.axis_index("s")
    base = wid * _CH
    last = wid == _NW - 1

    @pl.when(jnp.logical_not(last))
    def _():
        pltpu.sync_copy(xh.at[pl.ds(base, _CH)], xv)
        pltpu.sync_copy(yh.at[pl.ds(base, _CH)], yv)
        pltpu.sync_copy(zh.at[pl.ds(base, _CH)], zv)

    @pl.when(last)
    def _():
        pltpu.sync_copy(xh.at[pl.ds(base, _TAIL)], xv.at[pl.ds(0, _TAIL)])
        pltpu.sync_copy(yh.at[pl.ds(base, _TAIL)], yv.at[pl.ds(0, _TAIL)])
        pltpu.sync_copy(zh.at[pl.ds(base, _TAIL)], zv.at[pl.ds(0, _TAIL)])

    pltpu.sync_copy(ph, pv)
    pvec = pv[...]
    mnx, mny, mnz = pvec[0], pvec[1], pvec[2]
    gex, gey, gez = pvec[3], pvec[4], pvec[5]
    inv_sx, inv_sy, inv_sz = pvec[6], pvec[7], pvec[8]
    inf_f = f32(jnp.inf)
    iotav = jnp.arange(16, dtype=i32)

    def center(idx_f, ge, mn):
        return (idx_f + f32(0.5)) * ge * f32(2.0) + mn

    def cmpex(v, i, j):
        lo = jnp.minimum(v[i], v[j])
        v[j] = jnp.maximum(v[i], v[j])
        v[i] = lo

    def sort16(v):
        for i, j in _BATCHER16:
            cmpex(v, i, j)
        return v

    def merge_into(best, ch):
        m = [jnp.minimum(best[i], ch[15 - i]) for i in range(16)]
        for k in (8, 4, 2, 1):
            for i in range(16):
                if i % (2 * k) < k:
                    cmpex(m, i, i + k)
        return m

    def group_body(g, carry):
        g16 = g * 16
        xg = xv[pl.ds(g16, 16)]
        yg = yv[pl.ds(g16, 16)]
        zg = zv[pl.ds(g16, 16)]
        bx = jnp.clip(((xg - mnx) * inv_sx - f32(0.5)).astype(i32) - 1, 0, 4)
        by = jnp.clip(((yg - mny) * inv_sy - f32(0.5)).astype(i32) - 1, 0, 4)
        bz = jnp.clip(((zg - mnz) * inv_sz - f32(0.5)).astype(i32) - 1, 0, 4)
        ptsq = xg * xg + yg * yg + zg * zg
        xx2 = xg + xg
        yy2 = yg + yg
        zz2 = zg + zg
        ax, ay, az, gxv, gyv, gzv = [], [], [], [], [], []
        for o in range(4):
            cx = center((bx + o).astype(f32), gex, mnx)
            cy = center((by + o).astype(f32), gey, mny)
            cz = center((bz + o).astype(f32), gez, mnz)
            ax.append(cx * (cx - xx2) + ptsq)
            ay.append(cy * (cy - yy2))
            az.append(cz * (cz - zz2))
            gxv.append((bx + o) * 64)
            gyv.append((by + o) * 8)
            gzv.append(bz + o)
        best = None
        for ox in range(4):
            chunk = []
            for oy in range(4):
                axy = ax[ox] + ay[oy]
                gxy = gxv[ox] + gyv[oy]
                for oz in range(4):
                    d2 = jnp.maximum(axy + az[oz], f32(0.0))
                    gi = gxy + gzv[oz]
                    key = ((lax.bitcast_convert_type(d2, u32) & u32(0xFFFFFE00))
                           | lax.bitcast_convert_type(gi, u32))
                    chunk.append(key)
            chunk = sort16(chunk)
            best = chunk if best is None else merge_into(best, chunk)
        def sqd(pg, b, ge, mn):
            dd = pg - center(b.astype(f32), ge, mn)
            return dd * dd

        e = jnp.minimum(
            jnp.minimum(
                jnp.minimum(jnp.where(bx > 0, sqd(xg, bx - 1, gex, mnx), inf_f),
                            jnp.where(bx < 4, sqd(xg, bx + 4, gex, mnx), inf_f)),
                jnp.minimum(jnp.where(by > 0, sqd(yg, by - 1, gey, mny), inf_f),
                            jnp.where(by < 4, sqd(yg, by + 4, gey, mny), inf_f))),
            jnp.minimum(jnp.where(bz > 0, sqd(zg, bz - 1, gez, mnz), inf_f),
                        jnp.where(bz < 4, sqd(zg, bz + 4, gez, mnz), inf_f)))
        ekey = lax.bitcast_convert_type(e, u32) & u32(0xFFFFFE00)
        safe = jnp.all(best[_A - 1] < ekey)

        def full_scan(args):
            xg, yg, zg, ptsq = args
            xx2 = xg + xg
            yy2 = yg + yg
            zz2 = zg + zg
            best = [jnp.full((16,), u32(0xFFFFFFFF)) for _ in range(_A)]

            def fb(r, best):
                chunk = []
                for t in range(16):
                    s = r * 16 + t
                    cx = center((s // 64).astype(f32), gex, mnx)
                    cy = center(((s // 8) % 8).astype(f32), gey, mny)
                    cz = center((s % 8).astype(f32), gez, mnz)
                    d2 = jnp.maximum(((cx * cx + cy * cy + cz * cz) + ptsq)
                                     - (cx * xx2 + cy * yy2 + cz * zz2), f32(0.0))
                    key = ((lax.bitcast_convert_type(d2, u32) & u32(0xFFFFFE00))
                           | u32(s))
                    chunk.append(key)
                return tuple(merge_into(list(best), sort16(chunk)))

            return lax.fori_loop(0, 32, fb, tuple(best))

        best = lax.cond(safe, lambda a: tuple(best), full_scan, (xg, yg, zg, ptsq))
        rows = (g16 + iotav) * _A
        for k in range(_A):
            plsc.store_scatter(ov, [rows + k],
                               lax.bitcast_convert_type(best[k] & u32(_S - 1), i32))
        return carry

    lax.fori_loop(0, _CH // 16, group_body, 0)

    @pl.when(jnp.logical_not(last))
    def _():
        pltpu.sync_copy(ov, out_h.at[pl.ds(base * _A, _CH * _A)])

    @pl.when(last)
    def _():
        pltpu.sync_copy(ov.at[pl.ds(0, _TAIL * _A)],
                        out_h.at[pl.ds(base * _A, _TAIL * _A)])


def kernel(point_pos):
    p = point_pos.shape[0]
    px_pos, mn, mx = pl.pallas_call(
        _grid_init_kernel,
        grid=(p // _RB,),
        in_specs=[pl.BlockSpec((_RB, _DIM), lambda i: (i, 0))],
        out_specs=(
            pl.BlockSpec((_S, _DIM), lambda i: (0, 0)),
            pl.BlockSpec((1, _DIM), lambda i: (0, 0)),
            pl.BlockSpec((1, _DIM), lambda i: (0, 0)),
        ),
        out_shape=(
            jax.ShapeDtypeStruct((_S, _DIM), jnp.float32),
            jax.ShapeDtypeStruct((1, _DIM), jnp.float32),
            jax.ShapeDtypeStruct((1, _DIM), jnp.float32),
        ),
    )(point_pos)

    mn1 = mn[0]
    ge = (mx[0] - mn1) / jnp.float32(_GRID) * jnp.float32(0.5)
    inv_step = jnp.float32(1.0) / (ge * jnp.float32(2.0))
    params = jnp.concatenate([mn1, ge, inv_step, jnp.zeros((7,), jnp.float32)])

    mesh = plsc.VectorSubcoreMesh(core_axis_name="c", subcore_axis_name="s")
    px_ids = pl.kernel(
        _sc_knn_body,
        out_type=jax.ShapeDtypeStruct((_P * _A,), jnp.int32),
        mesh=mesh,
        compiler_params=pltpu.CompilerParams(needs_layout_passes=False),
        scratch_types=[
            pltpu.VMEM((_CH,), jnp.float32),
            pltpu.VMEM((_CH,), jnp.float32),
            pltpu.VMEM((_CH,), jnp.float32),
            pltpu.VMEM((16,), jnp.float32),
            pltpu.VMEM((_CH * _A,), jnp.int32),
        ],
    )(point_pos[:, 0], point_pos[:, 1], point_pos[:, 2], params)

    pt_ids = jnp.repeat(jnp.arange(p, dtype=jnp.int32), _A)
    assoc = jnp.stack([pt_ids, px_ids], axis=-1)
    return px_pos, assoc


# speedup vs baseline: 1.1616x; 1.1616x over previous
import jax
import jax.numpy as jnp
from jax import lax
from jax.experimental import pallas as pl
from jax.experimental.pallas import tpu as pltpu
from jax.experimental.pallas import tpu_sc as plsc

_GRID = 8
_DIM = 3
_A = 16
_S = _GRID ** 3
_NW = 32
_CH = 3136
_P = 100000
_TAIL = _P - (_NW - 1) * _CH
_RB = 2000


def _batcher_pairs(n):
    pairs = []
    p = 1
    while p < n:
        k = p
        while k >= 1:
            for j in range(k % p, n - k, 2 * k):
                for i in range(0, min(k, n - j - k)):
                    if (i + j) // (2 * p) == (i + j + k) // (2 * p):
                        pairs.append((i + j, i + j + k))
            k //= 2
        p *= 2
    return pairs


_BATCHER16 = _batcher_pairs(16)


def _grid_init_kernel(pts_ref, px_ref, mn_ref, mx_ref):
    pid = pl.program_id(0)
    mn = jnp.min(pts_ref[...], axis=0, keepdims=True)
    mx = jnp.max(pts_ref[...], axis=0, keepdims=True)

    @pl.when(pid == 0)
    def _():
        mn_ref[...] = mn
        mx_ref[...] = mx

    @pl.when(pid != 0)
    def _():
        mn_ref[...] = jnp.minimum(mn_ref[...], mn)
        mx_ref[...] = jnp.maximum(mx_ref[...], mx)

    @pl.when(pid == pl.num_programs(0) - 1)
    def _():
        ge = (mx_ref[...] - mn_ref[...]) / jnp.float32(_GRID) * jnp.float32(0.5)
        s = lax.broadcasted_iota(jnp.int32, (_S, _DIM), 0)
        c = lax.broadcasted_iota(jnp.int32, (_S, _DIM), 1)
        mesh = jnp.where(c == 0, s // (_GRID * _GRID),
                         jnp.where(c == 1, (s // _GRID) % _GRID, s % _GRID))
        mesh_ph = mesh.astype(jnp.float32) + jnp.float32(0.5)
        px_ref[...] = mesh_ph * ge * jnp.float32(2.0) + mn_ref[...]


def _sc_knn_body(xh, yh, zh, ph, out_h, xv, yv, zv, pv, ov):
    f32 = jnp.float32
    i32 = jnp.int32
    u32 = jnp.uint32
    wid = lax.axis_index("c") * 16 + lax.axis_index("s")
    base = wid * _CH
    last = wid == _NW - 1

    @pl.when(jnp.logical_not(last))
    def _():
        pltpu.sync_copy(xh.at[pl.ds(base, _CH)], xv)
        pltpu.sync_copy(yh.at[pl.ds(base, _CH)], yv)
        pltpu.sync_copy(zh.at[pl.ds(base, _CH)], zv)

    @pl.when(last)
    def _():
        pltpu.sync_copy(xh.at[pl.ds(base, _TAIL)], xv.at[pl.ds(0, _TAIL)])
        pltpu.sync_copy(yh.at[pl.ds(base, _TAIL)], yv.at[pl.ds(0, _TAIL)])
        pltpu.sync_copy(zh.at[pl.ds(base, _TAIL)], zv.at[pl.ds(0, _TAIL)])

    pltpu.sync_copy(ph, pv)
    pvec = pv[...]
    mnx, mny, mnz = pvec[0], pvec[1], pvec[2]
    gex, gey, gez = pvec[3], pvec[4], pvec[5]
    inv_sx, inv_sy, inv_sz = pvec[6], pvec[7], pvec[8]
    inf_f = f32(jnp.inf)
    iotav = jnp.arange(16, dtype=i32)

    def center(idx_f, ge, mn):
        return (idx_f + f32(0.5)) * ge * f32(2.0) + mn

    def cmpex(v, i, j):
        lo = jnp.minimum(v[i], v[j])
        v[j] = jnp.maximum(v[i], v[j])
        v[i] = lo

    def sort16(v):
        for i, j in _BATCHER16:
            cmpex(v, i, j)
        return v

    def merge_into(best, ch):
        m = [jnp.minimum(best[i], ch[15 - i]) for i in range(16)]
        for k in (8, 4, 2, 1):
            for i in range(16):
                if i % (2 * k) < k:
                    cmpex(m, i, i + k)
        return m

    def group_body(gg, carry):
        for u in range(2):
            _do_group(gg * 2 + u)
        return carry

    def _do_group(g):
        g16 = g * 16
        xg = xv[pl.ds(g16, 16)]
        yg = yv[pl.ds(g16, 16)]
        zg = zv[pl.ds(g16, 16)]
        bx = jnp.clip(((xg - mnx) * inv_sx - f32(0.5)).astype(i32) - 1, 0, 4)
        by = jnp.clip(((yg - mny) * inv_sy - f32(0.5)).astype(i32) - 1, 0, 4)
        bz = jnp.clip(((zg - mnz) * inv_sz - f32(0.5)).astype(i32) - 1, 0, 4)
        ptsq = xg * xg + yg * yg + zg * zg
        xx2 = xg + xg
        yy2 = yg + yg
        zz2 = zg + zg
        ax, ay, az, gxv, gyv, gzv = [], [], [], [], [], []
        for o in range(4):
            cx = center((bx + o).astype(f32), gex, mnx)
            cy = center((by + o).astype(f32), gey, mny)
            cz = center((bz + o).astype(f32), gez, mnz)
            ax.append(cx * (cx - xx2) + ptsq)
            ay.append(cy * (cy - yy2))
            az.append(cz * (cz - zz2))
            gxv.append((bx + o) * 64)
            gyv.append((by + o) * 8)
            gzv.append(bz + o)
        best = None
        for ox in range(4):
            chunk = []
            for oy in range(4):
                axy = ax[ox] + ay[oy]
                gxy = gxv[ox] + gyv[oy]
                for oz in range(4):
                    d2 = jnp.maximum(axy + az[oz], f32(0.0))
                    gi = gxy + gzv[oz]
                    key = ((lax.bitcast_convert_type(d2, u32) & u32(0xFFFFFE00))
                           | lax.bitcast_convert_type(gi, u32))
                    chunk.append(key)
            chunk = sort16(chunk)
            best = chunk if best is None else merge_into(best, chunk)
        def sqd(pg, b, ge, mn):
            dd = pg - center(b.astype(f32), ge, mn)
            return dd * dd

        e = jnp.minimum(
            jnp.minimum(
                jnp.minimum(jnp.where(bx > 0, sqd(xg, bx - 1, gex, mnx), inf_f),
                            jnp.where(bx < 4, sqd(xg, bx + 4, gex, mnx), inf_f)),
                jnp.minimum(jnp.where(by > 0, sqd(yg, by - 1, gey, mny), inf_f),
                            jnp.where(by < 4, sqd(yg, by + 4, gey, mny), inf_f))),
            jnp.minimum(jnp.where(bz > 0, sqd(zg, bz - 1, gez, mnz), inf_f),
                        jnp.where(bz < 4, sqd(zg, bz + 4, gez, mnz), inf_f)))
        ekey = lax.bitcast_convert_type(e, u32) & u32(0xFFFFFE00)
        safe = jnp.all(best[_A - 1] < ekey)

        def full_scan(args):
            xg, yg, zg, ptsq = args
            xx2 = xg + xg
            yy2 = yg + yg
            zz2 = zg + zg
            best = [jnp.full((16,), u32(0xFFFFFFFF)) for _ in range(_A)]

            def fb(r, best):
                chunk = []
                for t in range(16):
                    s = r * 16 + t
                    cx = center((s // 64).astype(f32), gex, mnx)
                    cy = center(((s // 8) % 8).astype(f32), gey, mny)
                    cz = center((s % 8).astype(f32), gez, mnz)
                    d2 = jnp.maximum(((cx * cx + cy * cy + cz * cz) + ptsq)
                                     - (cx * xx2 + cy * yy2 + cz * zz2), f32(0.0))
                    key = ((lax.bitcast_convert_type(d2, u32) & u32(0xFFFFFE00))
                           | u32(s))
                    chunk.append(key)
                return tuple(merge_into(list(best), sort16(chunk)))

            return lax.fori_loop(0, 32, fb, tuple(best))

        best = lax.cond(safe, lambda a: tuple(best), full_scan, (xg, yg, zg, ptsq))
        rows = (g16 + iotav) * _A
        for k in range(_A):
            plsc.store_scatter(ov, [rows + k],
                               lax.bitcast_convert_type(best[k] & u32(_S - 1), i32))

    lax.fori_loop(0, _CH // 32, group_body, 0)

    @pl.when(jnp.logical_not(last))
    def _():
        pltpu.sync_copy(ov, out_h.at[pl.ds(base * _A, _CH * _A)])

    @pl.when(last)
    def _():
        pltpu.sync_copy(ov.at[pl.ds(0, _TAIL * _A)],
                        out_h.at[pl.ds(base * _A, _TAIL * _A)])


def kernel(point_pos):
    p = point_pos.shape[0]
    px_pos, mn, mx = pl.pallas_call(
        _grid_init_kernel,
        grid=(p // _RB,),
        in_specs=[pl.BlockSpec((_RB, _DIM), lambda i: (i, 0))],
        out_specs=(
            pl.BlockSpec((_S, _DIM), lambda i: (0, 0)),
            pl.BlockSpec((1, _DIM), lambda i: (0, 0)),
            pl.BlockSpec((1, _DIM), lambda i: (0, 0)),
        ),
        out_shape=(
            jax.ShapeDtypeStruct((_S, _DIM), jnp.float32),
            jax.ShapeDtypeStruct((1, _DIM), jnp.float32),
            jax.ShapeDtypeStruct((1, _DIM), jnp.float32),
        ),
    )(point_pos)

    mn1 = mn[0]
    ge = (mx[0] - mn1) / jnp.float32(_GRID) * jnp.float32(0.5)
    inv_step = jnp.float32(1.0) / (ge * jnp.float32(2.0))
    params = jnp.concatenate([mn1, ge, inv_step, jnp.zeros((7,), jnp.float32)])

    mesh = plsc.VectorSubcoreMesh(core_axis_name="c", subcore_axis_name="s")
    px_ids = pl.kernel(
        _sc_knn_body,
        out_type=jax.ShapeDtypeStruct((_P * _A,), jnp.int32),
        mesh=mesh,
        compiler_params=pltpu.CompilerParams(needs_layout_passes=False),
        scratch_types=[
            pltpu.VMEM((_CH,), jnp.float32),
            pltpu.VMEM((_CH,), jnp.float32),
            pltpu.VMEM((_CH,), jnp.float32),
            pltpu.VMEM((16,), jnp.float32),
            pltpu.VMEM((_CH * _A,), jnp.int32),
        ],
    )(point_pos[:, 0], point_pos[:, 1], point_pos[:, 2], params)

    pt_ids = jnp.repeat(jnp.arange(p, dtype=jnp.int32), _A)
    assoc = jnp.stack([pt_ids, px_ids], axis=-1)
    return px_pos, assoc
